# baseline (device time: 125800 ns/iter reference)
import jax
import jax.numpy as jnp
import numpy as np
from jax import lax
from jax.experimental import pallas as pl
from jax.experimental.pallas import tpu as pltpu

N_DEV = 4
S = 1024
D = 1024
HQ = 8
HH = 4
DH = 128
DHALF = HH * DH
SCALE = 0.08838834764831843
NEG = -1e9
BLK = 64
NBLK = S // BLK

C0 = [b for b in range(NBLK) if b % 3 == 0]
C1 = [b for b in range(NBLK) if b % 3 == 1]
C2 = [b for b in range(NBLK) if b % 3 == 2]
ROW_ORDER = C0 + C1 + C2
N0 = len(C0) * BLK
N12 = (len(C1) + len(C2)) * BLK
_KBASE = {0: 0, 1: N0, 2: N0 + len(C1) * BLK}


_G_OF = {b: g for g, b in enumerate(ROW_ORDER)}

def _bias12_np():
    r = np.arange(N12)[:, None]
    c = np.arange(N12)[None, :]
    same_half = (r // 320) == (c // 320)
    diff_seg = ((r % 320) // BLK) != ((c % 320) // BLK)
    return np.where(same_half & diff_seg, NEG, 0.0)


def kernel(x, Wq, K_ext, V_ext, Wo):
    xb = x.reshape(S, D).astype(jnp.bfloat16)
    xb = jnp.concatenate(
        [xb[b * BLK:(b + 1) * BLK] for b in ROW_ORDER], axis=0)
    Wq16 = Wq.astype(jnp.bfloat16)
    Wo16 = Wo.astype(jnp.bfloat16)
    bias12 = jnp.asarray(_bias12_np(), dtype=jnp.bfloat16)

    def body(x_ref, wq_ref, wo_ref, k_hbm, v_hbm, bias_ref, out_ref,
             sqr, sor, sql, sol, kst, vst,
             q_ref, ctx_ref, kvb_ref, vvb_ref,
             qr_s, qr_r, or_s, or_r, ql_s, ql_r, ol_s, ol_r, ksem, vsem):
        my_pos = lax.axis_index("i")
        right = lax.rem(my_pos + 1, N_DEV)
        left = lax.rem(my_pos + N_DEV - 1, N_DEV)

        def head_off(s):
            t = s // 2
            if s % 2 == 0:
                o = lax.rem(my_pos - t + N_DEV, N_DEV)
                return o * HQ
            o = lax.rem(my_pos + t, N_DEV)
            return o * HQ + HH

        fetches = {}

        def start_fetch(s):
            off = head_off(s)
            ds = []
            for b in range(NBLK):
                dst = _KBASE[b % 3] + (b // 3) * BLK
                kd = pltpu.make_async_copy(
                    k_hbm.at[my_pos, pl.ds(b * BLK, BLK), pl.ds(off, HH), :],
                    kst.at[pl.ds(dst, BLK)],
                    ksem.at[b])
                vd = pltpu.make_async_copy(
                    v_hbm.at[my_pos, pl.ds(b * BLK, BLK), pl.ds(off, HH), :],
                    vst.at[pl.ds(dst, BLK)],
                    vsem.at[b])
                kd.start()
                vd.start()
                ds.append(kd)
                ds.append(vd)
            fetches[s] = ds

        start_fetch(0)

        barrier = pltpu.get_barrier_semaphore()
        for nbr in (left, right):
            pl.semaphore_signal(
                barrier, inc=1, device_id=(nbr,),
                device_id_type=pl.DeviceIdType.MESH,
            )
        pl.semaphore_wait(barrier, 2)

        bias12 = bias_ref[...]

        def dotT(a, b):
            return lax.dot_general(
                a, b, (((1,), (1,)), ((), ())),
                preferred_element_type=jnp.float32)

        def compute_half(s, wq_h, wo_h):
            for dma in fetches[s]:
                dma.wait()
            kvb_ref[...] = kst[...].reshape(S, DHALF).astype(jnp.bfloat16)
            vvb_ref[...] = vst[...].reshape(S, DHALF).astype(jnp.bfloat16)
            if s + 1 < 2 * N_DEV:
                start_fetch(s + 1)
            q_ref[...] = (jnp.dot(x_ref[...], wq_h,
                                  preferred_element_type=jnp.float32)
                          * SCALE).astype(jnp.bfloat16)
            for h in range(HH):
                qh = q_ref[:, h * DH:(h + 1) * DH]
                kh = kvb_ref[:, h * DH:(h + 1) * DH]
                vh = vvb_ref[:, h * DH:(h + 1) * DH]
                s0 = dotT(qh[0:N0], kh[0:N0])
                e0 = jnp.exp(s0)
                r0 = 1.0 / jnp.sum(e0, axis=1, keepdims=True)
                c0 = jnp.dot(e0.astype(jnp.bfloat16), vh[0:N0],
                             preferred_element_type=jnp.float32) * r0
                q12 = qh[N0:S]
                sa = dotT(q12, kh[0:BLK])
                s12 = dotT(q12, kh[N0:S]) + bias12.astype(jnp.float32)
                ea = jnp.exp(sa)
                e12 = jnp.exp(s12)
                rs = 1.0 / (jnp.sum(ea, axis=1, keepdims=True)
                            + jnp.sum(e12, axis=1, keepdims=True))
                c12 = (jnp.dot(ea.astype(jnp.bfloat16), vh[0:BLK],
                               preferred_element_type=jnp.float32)
                       + jnp.dot(e12.astype(jnp.bfloat16), vh[N0:S],
                                 preferred_element_type=jnp.float32)) * rs
                ctx_ref[0:N0, h * DH:(h + 1) * DH] = c0.astype(jnp.bfloat16)
                ctx_ref[N0:S, h * DH:(h + 1) * DH] = c12.astype(jnp.bfloat16)
            return jnp.dot(ctx_ref[...], wo_h,
                           preferred_element_type=jnp.float32)

        def mk(slots, ssem, rsem, h, src, dev):
            return pltpu.make_async_remote_copy(
                src_ref=src, dst_ref=slots.at[h], send_sem=ssem.at[h],
                recv_sem=rsem.at[h], device_id=(dev,),
                device_id_type=pl.DeviceIdType.MESH,
            )

        rq = mk(sqr, qr_s, qr_r, 0, wq_ref.at[:, 0:DHALF], right)
        ro = mk(sor, or_s, or_r, 0, wo_ref.at[0:DHALF, :], right)
        lq = mk(sql, ql_s, ql_r, 0, wq_ref.at[:, DHALF:D], left)
        lo = mk(sol, ol_s, ol_r, 0, wo_ref.at[DHALF:D, :], left)
        for dma in (rq, ro, lq, lo):
            dma.start()
        out_ref[...] = compute_half(0, wq_ref[:, 0:DHALF], wo_ref[0:DHALF, :])
        out_ref[...] += compute_half(1, wq_ref[:, DHALF:D], wo_ref[DHALF:D, :])
        for h in range(1, N_DEV):
            rq.wait()
            ro.wait()
            if h < N_DEV - 1:
                rq = mk(sqr, qr_s, qr_r, h, sqr.at[h - 1], right)
                ro = mk(sor, or_s, or_r, h, sor.at[h - 1], right)
                rq.start()
                ro.start()
            out_ref[...] += compute_half(2 * h, sqr[h - 1], sor[h - 1])
            lq.wait()
            lo.wait()
            if h < N_DEV - 1:
                lq = mk(sql, ql_s, ql_r, h, sql.at[h - 1], left)
                lo = mk(sol, ol_s, ol_r, h, sol.at[h - 1], left)
                lq.start()
                lo.start()
            out_ref[...] += compute_half(2 * h + 1, sql[h - 1], sol[h - 1])

    nh = N_DEV - 1
    out = pl.pallas_call(
        body,
        out_shape=jax.ShapeDtypeStruct((S, D), jnp.float32),
        in_specs=[pl.BlockSpec(memory_space=pltpu.VMEM)] * 3
        + [pl.BlockSpec(memory_space=pl.ANY)] * 2
        + [pl.BlockSpec(memory_space=pltpu.VMEM)],
        out_specs=pl.BlockSpec(memory_space=pltpu.VMEM),
        scratch_shapes=[
            pltpu.VMEM((nh, D, DHALF), jnp.bfloat16),
            pltpu.VMEM((nh, DHALF, D), jnp.bfloat16),
            pltpu.VMEM((nh, D, DHALF), jnp.bfloat16),
            pltpu.VMEM((nh, DHALF, D), jnp.bfloat16),
            pltpu.VMEM((S, HH, DH), jnp.float32),
            pltpu.VMEM((S, HH, DH), jnp.float32),
            pltpu.VMEM((S, DHALF), jnp.bfloat16),
            pltpu.VMEM((S, DHALF), jnp.bfloat16),
            pltpu.VMEM((S, DHALF), jnp.bfloat16),
            pltpu.VMEM((S, DHALF), jnp.bfloat16),
        ] + [pltpu.SemaphoreType.DMA((nh,))] * 8
        + [pltpu.SemaphoreType.DMA((NBLK,))] * 2,
        compiler_params=pltpu.CompilerParams(
            collective_id=0, vmem_limit_bytes=56 * 1024 * 1024
        ),
    )(xb, Wq16, Wo16, K_ext, V_ext, bias12)
    out = jnp.concatenate(
        [out[_G_OF[b] * BLK:(_G_OF[b] + 1) * BLK] for b in range(NBLK)],
        axis=0)
    return out.reshape(1, S, D)


# device time: 114636 ns/iter; 1.0974x vs baseline; 1.0974x over previous
import jax
import jax.numpy as jnp
from jax import lax
from jax.experimental import pallas as pl
from jax.experimental.pallas import tpu as pltpu

N_DEV = 4
S = 1024
D = 1024
HQ = 8
HH = 4
DH = 128
DHALF = HH * DH
SCALE = 0.08838834764831843
NEG = -1e9


def kernel(x, Wq, K_ext, V_ext, Wo):
    xb = x.reshape(S, D).astype(jnp.bfloat16)
    Wq16 = Wq.astype(jnp.bfloat16)
    Wo16 = Wo.astype(jnp.bfloat16)

    def body(x_ref, wq_ref, wo_ref, k_hbm, v_hbm, out_ref,
             sqr, sor, sql, sol, kst, vst,
             qr_s, qr_r, or_s, or_r, ql_s, ql_r, ol_s, ol_r, ksem, vsem):
        my_pos = lax.axis_index("i")
        right = lax.rem(my_pos + 1, N_DEV)
        left = lax.rem(my_pos + N_DEV - 1, N_DEV)

        def head_off(s):
            t = s // 2
            if s % 2 == 0:
                o = lax.rem(my_pos - t + N_DEV, N_DEV)
                return o * HQ
            o = lax.rem(my_pos + t, N_DEV)
            return o * HQ + HH

        fetches = {}

        def start_fetch(s):
            off = head_off(s)
            slot = s % 2
            kd = pltpu.make_async_copy(
                k_hbm.at[my_pos, :, pl.ds(off, HH), :], kst.at[slot],
                ksem.at[slot])
            vd = pltpu.make_async_copy(
                v_hbm.at[my_pos, :, pl.ds(off, HH), :], vst.at[slot],
                vsem.at[slot])
            kd.start()
            vd.start()
            fetches[s] = (kd, vd)

        start_fetch(0)
        start_fetch(1)

        barrier = pltpu.get_barrier_semaphore()
        for nbr in (left, right):
            pl.semaphore_signal(
                barrier, inc=1, device_id=(nbr,),
                device_id_type=pl.DeviceIdType.MESH,
            )
        pl.semaphore_wait(barrier, 2)

        qb = lax.broadcasted_iota(jnp.int32, (S, S), 0) // 64
        kb = lax.broadcasted_iota(jnp.int32, (S, S), 1) // 64
        mask = (qb == kb) | (kb == 0) | ((qb + kb) % 3 == 0)
        bias = jnp.where(mask, 0.0, NEG).astype(jnp.float32)

        xv = x_ref[...]

        def compute_half(s, wq_h, wo_h):
            slot = s % 2
            kd, vd = fetches[s]
            kd.wait()
            vd.wait()
            kv = kst[slot].reshape(S, DHALF).astype(jnp.bfloat16)
            vv = vst[slot].reshape(S, DHALF).astype(jnp.bfloat16)
            if s + 2 < 2 * N_DEV:
                start_fetch(s + 2)
            q = (jnp.dot(xv, wq_h, preferred_element_type=jnp.float32)
                 * SCALE).astype(jnp.bfloat16)
            ctx_cols = []
            for h in range(HH):
                qh = q[:, h * DH:(h + 1) * DH]
                kh = kv[:, h * DH:(h + 1) * DH]
                sc = lax.dot_general(
                    qh, kh, (((1,), (1,)), ((), ())),
                    preferred_element_type=jnp.float32,
                ) + bias
                e = jnp.exp(sc)
                rs = 1.0 / jnp.sum(e, axis=1, keepdims=True)
                vh = vv[:, h * DH:(h + 1) * DH]
                ch = jnp.dot(e.astype(jnp.bfloat16), vh,
                             preferred_element_type=jnp.float32) * rs
                ctx_cols.append(ch.astype(jnp.bfloat16))
            ctx = jnp.concatenate(ctx_cols, axis=1)
            return jnp.dot(ctx, wo_h, preferred_element_type=jnp.float32)

        def mk(slots, ssem, rsem, h, src, dev):
            return pltpu.make_async_remote_copy(
                src_ref=src, dst_ref=slots.at[h], send_sem=ssem.at[h],
                recv_sem=rsem.at[h], device_id=(dev,),
                device_id_type=pl.DeviceIdType.MESH,
            )

        rq = mk(sqr, qr_s, qr_r, 0, wq_ref.at[:, 0:DHALF], right)
        ro = mk(sor, or_s, or_r, 0, wo_ref.at[0:DHALF, :], right)
        lq = mk(sql, ql_s, ql_r, 0, wq_ref.at[:, DHALF:D], left)
        lo = mk(sol, ol_s, ol_r, 0, wo_ref.at[DHALF:D, :], left)
        for dma in (rq, ro, lq, lo):
            dma.start()
        out_ref[...] = compute_half(0, wq_ref[:, 0:DHALF], wo_ref[0:DHALF, :])
        out_ref[...] += compute_half(1, wq_ref[:, DHALF:D], wo_ref[DHALF:D, :])
        for h in range(1, N_DEV):
            rq.wait()
            ro.wait()
            lq.wait()
            lo.wait()
            if h < N_DEV - 1:
                rq = mk(sqr, qr_s, qr_r, h, sqr.at[h - 1], right)
                ro = mk(sor, or_s, or_r, h, sor.at[h - 1], right)
                lq = mk(sql, ql_s, ql_r, h, sql.at[h - 1], left)
                lo = mk(sol, ol_s, ol_r, h, sol.at[h - 1], left)
                rq.start()
                ro.start()
                lq.start()
                lo.start()
            out_ref[...] += compute_half(2 * h, sqr[h - 1], sor[h - 1])
            out_ref[...] += compute_half(2 * h + 1, sql[h - 1], sol[h - 1])

    nh = N_DEV - 1
    out = pl.pallas_call(
        body,
        out_shape=jax.ShapeDtypeStruct((S, D), jnp.float32),
        in_specs=[pl.BlockSpec(memory_space=pltpu.VMEM)] * 3
        + [pl.BlockSpec(memory_space=pl.ANY)] * 2,
        out_specs=pl.BlockSpec(memory_space=pltpu.VMEM),
        scratch_shapes=[
            pltpu.VMEM((nh, D, DHALF), jnp.bfloat16),
            pltpu.VMEM((nh, DHALF, D), jnp.bfloat16),
            pltpu.VMEM((nh, D, DHALF), jnp.bfloat16),
            pltpu.VMEM((nh, DHALF, D), jnp.bfloat16),
            pltpu.VMEM((2, S, HH, DH), jnp.float32),
            pltpu.VMEM((2, S, HH, DH), jnp.float32),
        ] + [pltpu.SemaphoreType.DMA((nh,))] * 8
        + [pltpu.SemaphoreType.DMA((2,))] * 2,
        compiler_params=pltpu.CompilerParams(
            collective_id=0, vmem_limit_bytes=56 * 1024 * 1024
        ),
    )(xb, Wq16, Wo16, K_ext, V_ext)
    return out.reshape(1, S, D)


# device time: 114231 ns/iter; 1.1013x vs baseline; 1.0035x over previous
import jax
import jax.numpy as jnp
from jax import lax
from jax.experimental import pallas as pl
from jax.experimental.pallas import tpu as pltpu

N_DEV = 4
S = 1024
D = 1024
HQ = 8
HH = 4
DH = 128
DHALF = HH * DH
SCALE = 0.08838834764831843
NEG = -1e9


def kernel(x, Wq, K_ext, V_ext, Wo):
    xb = x.reshape(S, D).astype(jnp.bfloat16)
    Wq16 = (Wq * SCALE).astype(jnp.bfloat16)
    Wo16 = Wo.astype(jnp.bfloat16)

    def body(x_ref, wq_ref, wo_ref, k_hbm, v_hbm, out_ref,
             sqr, sor, sql, sol, kst, vst,
             qr_s, qr_r, or_s, or_r, ql_s, ql_r, ol_s, ol_r, ksem, vsem):
        my_pos = lax.axis_index("i")
        right = lax.rem(my_pos + 1, N_DEV)
        left = lax.rem(my_pos + N_DEV - 1, N_DEV)

        def head_off(s):
            t = s // 2
            if s % 2 == 0:
                o = lax.rem(my_pos - t + N_DEV, N_DEV)
                return o * HQ
            o = lax.rem(my_pos + t, N_DEV)
            return o * HQ + HH

        fetches = {}

        def start_fetch(s):
            off = head_off(s)
            slot = s % 2
            kd = pltpu.make_async_copy(
                k_hbm.at[my_pos, :, pl.ds(off, HH), :], kst.at[slot],
                ksem.at[slot])
            vd = pltpu.make_async_copy(
                v_hbm.at[my_pos, :, pl.ds(off, HH), :], vst.at[slot],
                vsem.at[slot])
            kd.start()
            vd.start()
            fetches[s] = (kd, vd)

        start_fetch(0)
        start_fetch(1)

        barrier = pltpu.get_barrier_semaphore()
        for nbr in (left, right):
            pl.semaphore_signal(
                barrier, inc=1, device_id=(nbr,),
                device_id_type=pl.DeviceIdType.MESH,
            )
        pl.semaphore_wait(barrier, 2)

        qb = lax.broadcasted_iota(jnp.int32, (S, S), 0) // 64
        kb = lax.broadcasted_iota(jnp.int32, (S, S), 1) // 64
        mask = (qb == kb) | (kb == 0) | ((qb + kb) % 3 == 0)
        bias = jnp.where(mask, 0.0, NEG).astype(jnp.bfloat16)

        xv = x_ref[...]

        def compute_half(s, wq_h, wo_h):
            slot = s % 2
            kd, vd = fetches[s]
            kd.wait()
            vd.wait()
            kv = kst[slot].reshape(S, DHALF).astype(jnp.bfloat16)
            vv = vst[slot].reshape(S, DHALF).astype(jnp.bfloat16)
            if s + 2 < 2 * N_DEV:
                start_fetch(s + 2)
            q = jnp.dot(xv, wq_h,
                        preferred_element_type=jnp.float32
                        ).astype(jnp.bfloat16)
            ctx_cols = []
            for h in range(HH):
                qh = q[:, h * DH:(h + 1) * DH]
                kh = kv[:, h * DH:(h + 1) * DH]
                sc = lax.dot_general(
                    qh, kh, (((1,), (1,)), ((), ())),
                    preferred_element_type=jnp.float32,
                ).astype(jnp.bfloat16) + bias
                e = jnp.exp(sc)
                rs = 1.0 / jnp.sum(e, axis=1, keepdims=True,
                                   dtype=jnp.float32)
                vh = vv[:, h * DH:(h + 1) * DH]
                ch = jnp.dot(e, vh,
                             preferred_element_type=jnp.float32) * rs
                ctx_cols.append(ch.astype(jnp.bfloat16))
            ctx = jnp.concatenate(ctx_cols, axis=1)
            return jnp.dot(ctx, wo_h, preferred_element_type=jnp.float32)

        def mk(slots, ssem, rsem, h, src, dev):
            return pltpu.make_async_remote_copy(
                src_ref=src, dst_ref=slots.at[h], send_sem=ssem.at[h],
                recv_sem=rsem.at[h], device_id=(dev,),
                device_id_type=pl.DeviceIdType.MESH,
            )

        rq = mk(sqr, qr_s, qr_r, 0, wq_ref.at[:, 0:DHALF], right)
        ro = mk(sor, or_s, or_r, 0, wo_ref.at[0:DHALF, :], right)
        lq = mk(sql, ql_s, ql_r, 0, wq_ref.at[:, DHALF:D], left)
        lo = mk(sol, ol_s, ol_r, 0, wo_ref.at[DHALF:D, :], left)
        for dma in (rq, ro, lq, lo):
            dma.start()
        out_ref[...] = compute_half(0, wq_ref[:, 0:DHALF], wo_ref[0:DHALF, :])
        out_ref[...] += compute_half(1, wq_ref[:, DHALF:D], wo_ref[DHALF:D, :])
        for h in range(1, N_DEV):
            rq.wait()
            ro.wait()
            lq.wait()
            lo.wait()
            if h < N_DEV - 1:
                rq = mk(sqr, qr_s, qr_r, h, sqr.at[h - 1], right)
                ro = mk(sor, or_s, or_r, h, sor.at[h - 1], right)
                lq = mk(sql, ql_s, ql_r, h, sql.at[h - 1], left)
                lo = mk(sol, ol_s, ol_r, h, sol.at[h - 1], left)
                rq.start()
                ro.start()
                lq.start()
                lo.start()
            out_ref[...] += compute_half(2 * h, sqr[h - 1], sor[h - 1])
            out_ref[...] += compute_half(2 * h + 1, sql[h - 1], sol[h - 1])

    nh = N_DEV - 1
    out = pl.pallas_call(
        body,
        out_shape=jax.ShapeDtypeStruct((S, D), jnp.float32),
        in_specs=[pl.BlockSpec(memory_space=pltpu.VMEM)] * 3
        + [pl.BlockSpec(memory_space=pl.ANY)] * 2,
        out_specs=pl.BlockSpec(memory_space=pltpu.VMEM),
        scratch_shapes=[
            pltpu.VMEM((nh, D, DHALF), jnp.bfloat16),
            pltpu.VMEM((nh, DHALF, D), jnp.bfloat16),
            pltpu.VMEM((nh, D, DHALF), jnp.bfloat16),
            pltpu.VMEM((nh, DHALF, D), jnp.bfloat16),
            pltpu.VMEM((2, S, HH, DH), jnp.float32),
            pltpu.VMEM((2, S, HH, DH), jnp.float32),
        ] + [pltpu.SemaphoreType.DMA((nh,))] * 8
        + [pltpu.SemaphoreType.DMA((2,))] * 2,
        compiler_params=pltpu.CompilerParams(
            collective_id=0, vmem_limit_bytes=56 * 1024 * 1024
        ),
    )(xb, Wq16, Wo16, K_ext, V_ext)
    return out.reshape(1, S, D)
